# baseline (speedup 1.0000x reference)
import functools

import jax
import jax.numpy as jnp
from jax import lax
from jax.experimental import pallas as pl
from jax.experimental.pallas import tpu as pltpu
from jax.experimental.pallas import tpu_sc as plsc

N = 320000
D = 128
S = 10000
NC = 2
NS = 16
NW = NC * NS
CHUNK = 80
NCH_TOT = N // CHUNK
KPT = NCH_TOT // NW
NPAIR = (KPT - 1) // 2
STRIPE = 624
TAIL = S - NS * STRIPE

RB = 6400


def _scores_body(x_ref, w1_ref, b1_ref, w2_ref, b2_ref, ex_ref):
    h = jnp.tanh(
        jnp.dot(x_ref[...], w1_ref[...], preferred_element_type=jnp.float32)
        + b1_ref[...]
    )
    s = jnp.dot(h, w2_ref[...], preferred_element_type=jnp.float32) + b2_ref[...]
    ex_ref[...] = jnp.exp(s).reshape(1, RB // 128, 128)


def _scores(x, W1, b1, W2, b2):
    return pl.pallas_call(
        _scores_body,
        grid=(N // RB,),
        in_specs=[
            pl.BlockSpec((RB, D), lambda i: (i, 0)),
            pl.BlockSpec((D, D // 2), lambda i: (0, 0)),
            pl.BlockSpec((1, D // 2), lambda i: (0, 0)),
            pl.BlockSpec((D // 2, 1), lambda i: (0, 0)),
            pl.BlockSpec((1, 1), lambda i: (0, 0)),
        ],
        out_specs=pl.BlockSpec((1, RB // 128, 128), lambda i: (i, 0, 0)),
        out_shape=jax.ShapeDtypeStruct((N // RB, RB // 128, 128), jnp.float32),
    )(x, W1, b1.reshape(1, -1), W2, b2.reshape(1, 1))


def _pool_body(x_hbm, ex_hbm, ids_hbm, acc_out, den_out,
               xbuf, wbuf, exbuf, wexbuf, idbuf, sidbuf, denstage, cids,
               acc_sp, den_sp,
               insem0, insem1, outsem0, outsem1):
    c = lax.axis_index("c")
    sid = lax.axis_index("s")
    wid = c * NS + sid
    r0 = sid * STRIPE
    insem = (insem0, insem1)
    outsem = (outsem0, outsem1)

    def _zero_row(i, _):
        for j in range(D // 16):
            wbuf[0, i, pl.ds(j * 16, 16)] = jnp.zeros((16,), jnp.float32)
        return 0

    lax.fori_loop(0, CHUNK, _zero_row, 0)
    for g in range(STRIPE // 16):
        denstage[pl.ds(g * 16, 16)] = jnp.zeros((16,), jnp.float32)

    z2d = wbuf.at[0]
    nfull = STRIPE // CHUNK
    for part in range(nfull):
        pltpu.sync_copy(z2d, acc_sp.at[pl.ds(r0 + part * CHUNK, CHUNK)])
    pltpu.sync_copy(z2d.at[pl.ds(0, STRIPE - nfull * CHUNK)],
                    acc_sp.at[pl.ds(r0 + nfull * CHUNK,
                                    STRIPE - nfull * CHUNK)])
    pltpu.sync_copy(denstage, den_sp.at[pl.ds(r0, STRIPE)])

    @pl.when(sid == 0)
    def _zero_tail():
        pltpu.sync_copy(z2d.at[pl.ds(0, TAIL)],
                        acc_sp.at[pl.ds(NS * STRIPE, TAIL)])
        pltpu.sync_copy(denstage.at[pl.ds(0, TAIL)],
                        den_sp.at[pl.ds(NS * STRIPE, TAIL)])

    plsc.subcore_barrier()

    def _issue_in(b, k):
        ci = k * NW + wid
        r = ci * CHUNK
        pltpu.async_copy(x_hbm.at[pl.ds(r, CHUNK)], xbuf.at[b], insem[b])
        pltpu.async_copy(ex_hbm.at[pl.ds(r, CHUNK)], exbuf.at[b], insem[b])
        pltpu.async_copy(ids_hbm.at[pl.ds(r, CHUNK)],
                         idbuf.at[b, 0, pl.ds(16, CHUNK)], insem[b])

    def _wait_in(b, k):
        ci = k * NW + wid
        r = ci * CHUNK
        pltpu.make_async_copy(x_hbm.at[pl.ds(r, CHUNK)], xbuf.at[b],
                              insem[b]).wait()
        pltpu.make_async_copy(ex_hbm.at[pl.ds(r, CHUNK)], exbuf.at[b],
                              insem[b]).wait()
        pltpu.make_async_copy(ids_hbm.at[pl.ds(r, CHUNK)],
                              idbuf.at[b, 0, pl.ds(16, CHUNK)],
                              insem[b]).wait()

    def _compute(b):
        idbuf[b, 0, pl.ds(0, 16)] = jnp.full((16,), -1, jnp.int32)
        idbuf[b, 0, pl.ds(16 + CHUNK, 16)] = jnp.full((16,), -2, jnp.int32)

        def _grp(g, carry):
            pos0, *accs = carry
            iv = idbuf[b, 0, pl.ds(16 + g * 16, 16)]
            iv_sh = idbuf[b, 0, pl.ds(15 + g * 16, 16)]
            sidbuf[b, 0, pl.ds(g * 16, 16)] = iv
            ev = exbuf[b, pl.ds(g * 16, 16)]
            wexbuf[b, pl.ds(g * 16, 16)] = ev
            ndiff = jnp.where(iv == iv_sh, 0, 1)
            accs = list(accs)
            pos = pos0
            for q in range(16):
                i = g * 16 + q
                nd = ndiff[q]
                pos = pos + nd
                cids[b, pos] = iv[q]
                e = ev[q]
                ext = nd == 0
                for j in range(D // 16):
                    w = xbuf[b, i, pl.ds(j * 16, 16)] * e
                    accs[j] = jnp.where(ext, accs[j] + w, w)
                    wbuf[b, pos, pl.ds(j * 16, 16)] = accs[j]
            return (pos, *accs)

        z = jnp.zeros((16,), jnp.float32)
        carry = lax.fori_loop(0, CHUNK // 16, _grp,
                              (jnp.int32(-1),) + (z,) * (D // 16))
        pos_end = carry[0]
        m = (pos_end >> 4) + 1

        def _fill(i2, _):
            cids[b, i2] = S + (i2 & 15)
            return 0

        lax.fori_loop(pos_end + 1, m * 16, _fill, 0)
        return m

    iota16 = lax.iota(jnp.int32, 16)

    def _cidvec(b, t):
        v = jnp.zeros((16,), jnp.int32)
        for l in range(16):
            v = jnp.where(iota16 == l, cids[b, t * 16 + l], v)
        return v

    def _issue_out(b, m):
        for t in range(CHUNK // 16):
            @pl.when(t < m)
            def _():
                pltpu.async_copy(wbuf.at[b, pl.ds(t * 16, 16)],
                                 acc_sp.at[_cidvec(b, t)],
                                 outsem[b], add=True)
        pltpu.async_copy(wexbuf.at[b], den_sp.at[sidbuf.at[b, 0]],
                         outsem[b], add=True)

    def _wait_out(b, m):
        for t in range(CHUNK // 16):
            @pl.when(t < m)
            def _():
                pltpu.make_async_copy(wbuf.at[b, pl.ds(t * 16, 16)],
                                      acc_sp.at[iota16],
                                      outsem[b]).wait()
        pltpu.make_async_copy(wexbuf.at[b], den_sp.at[sidbuf.at[b, 0]],
                              outsem[b]).wait()

    _issue_in(0, 0)
    _issue_in(1, 1)

    def _pair(p, carry):
        m0, m1 = carry
        ms = [m0, m1]
        for b in (0, 1):
            k0 = 2 * p
            k = k0 + b
            _wait_in(b, k)

            @pl.when(p >= 1)
            def _():
                _wait_out(b, ms[b])

            ms[b] = _compute(b)
            _issue_out(b, ms[b])

            @pl.when(k + 2 <= KPT - 1)
            def _():
                _issue_in(b, k + 2)
        return (ms[0], ms[1])

    m0, m1 = lax.fori_loop(0, NPAIR, _pair, (jnp.int32(0), jnp.int32(0)))

    _wait_in(0, KPT - 1)
    _wait_out(0, m0)
    m0f = _compute(0)
    _issue_out(0, m0f)
    _wait_out(1, m1)
    _wait_out(0, m0f)

    plsc.subcore_barrier()
    pltpu.sync_copy(acc_sp.at[pl.ds(r0, STRIPE)],
                    acc_out.at[c, pl.ds(r0, STRIPE)])
    pltpu.sync_copy(den_sp.at[pl.ds(r0, STRIPE)], denstage)
    pltpu.sync_copy(denstage, den_out.at[pl.ds(c * S + r0, STRIPE)])

    @pl.when(sid == 0)
    def _copy_tail():
        pltpu.sync_copy(acc_sp.at[pl.ds(NS * STRIPE, TAIL)],
                        acc_out.at[c, pl.ds(NS * STRIPE, TAIL)])
        pltpu.sync_copy(den_sp.at[pl.ds(NS * STRIPE, TAIL)],
                        denstage.at[pl.ds(0, TAIL)])
        pltpu.sync_copy(denstage.at[pl.ds(0, TAIL)],
                        den_out.at[pl.ds(c * S + NS * STRIPE, TAIL)])


@functools.cache
def _pool():
    mesh = plsc.VectorSubcoreMesh(
        core_axis_name="c", subcore_axis_name="s",
        num_cores=NC, num_subcores=NS,
    )
    return pl.kernel(
        _pool_body,
        out_type=[
            jax.ShapeDtypeStruct((NC, S, D), jnp.float32),
            jax.ShapeDtypeStruct((NC * S,), jnp.float32),
        ],
        mesh=mesh,
        scratch_types=[
            pltpu.VMEM((2, CHUNK, D), jnp.float32),
            pltpu.VMEM((2, CHUNK, D), jnp.float32),
            pltpu.VMEM((2, CHUNK), jnp.float32),
            pltpu.VMEM((2, CHUNK), jnp.float32),
            pltpu.VMEM((2, 1, CHUNK + 32), jnp.int32),
            pltpu.VMEM((2, 1, CHUNK), jnp.int32),
            pltpu.VMEM((STRIPE,), jnp.float32),
            pltpu.SMEM((2, CHUNK + 16), jnp.int32),
            pltpu.VMEM_SHARED((S + 16, D), jnp.float32),
            pltpu.VMEM_SHARED((S,), jnp.float32),
            pltpu.SemaphoreType.DMA,
            pltpu.SemaphoreType.DMA,
            pltpu.SemaphoreType.DMA,
            pltpu.SemaphoreType.DMA,
        ],
    )


def _combine_body(acc_ref, den_ref, out_ref):
    den_all = den_ref[...]
    den = (den_all[0:S] + den_all[S:2 * S])[:, None]
    w = acc_ref[...][0] + acc_ref[...][1]
    safe = jnp.where(den > 0, den, 1.0)
    out_ref[...] = jnp.where(den > 0, w / safe, 0.0)


def _combine(acc, den):
    return pl.pallas_call(
        _combine_body,
        out_shape=jax.ShapeDtypeStruct((S, D), jnp.float32),
    )(acc, den)


def kernel(x, batch, W1, b1, W2, b2):
    ex = _scores(x, W1, b1, W2, b2)
    acc, den = _pool()(x, ex.reshape(N), batch)
    return _combine(acc, den)


# speedup vs baseline: 2.0266x; 2.0266x over previous
import functools

import jax
import jax.numpy as jnp
from jax import lax
from jax.experimental import pallas as pl
from jax.experimental.pallas import tpu as pltpu
from jax.experimental.pallas import tpu_sc as plsc

N = 320000
D = 128
S = 10000
NC = 2
NS = 16
NW = NC * NS
CHUNK = 80
N2 = N // 2
NCH_HALF = N2 // CHUNK
NPAIR = 31
STRIPE = 624
TAIL = S - NS * STRIPE

RB = 6400


def _scores_body(x_ref, w1_ref, b1_ref, w2_ref, b2_ref, ex_ref):
    h = jnp.tanh(
        jnp.dot(x_ref[...], w1_ref[...], preferred_element_type=jnp.float32)
        + b1_ref[...]
    )
    s = jnp.dot(h, w2_ref[...], preferred_element_type=jnp.float32) + b2_ref[...]
    ex_ref[...] = jnp.exp(s).reshape(1, RB // 128, 128)


def _scores(x, W1, b1, W2, b2):
    n = x.shape[0]
    return pl.pallas_call(
        _scores_body,
        grid=(n // RB,),
        in_specs=[
            pl.BlockSpec((RB, D), lambda i: (i, 0)),
            pl.BlockSpec((D, D // 2), lambda i: (0, 0)),
            pl.BlockSpec((1, D // 2), lambda i: (0, 0)),
            pl.BlockSpec((D // 2, 1), lambda i: (0, 0)),
            pl.BlockSpec((1, 1), lambda i: (0, 0)),
        ],
        out_specs=pl.BlockSpec((1, RB // 128, 128), lambda i: (i, 0, 0)),
        out_shape=jax.ShapeDtypeStruct((n // RB, RB // 128, 128), jnp.float32),
    )(x, W1, b1.reshape(1, -1), W2, b2.reshape(1, 1))


def _make_pool_body(h):
    base = h * N2

    def _pool_body(x_hbm, ex_hbm, ids_hbm, acc_out, den_out,
                   xbuf, wbuf, exbuf, wexbuf, idbuf, sidbuf, denstage,
                   acc_sp, den_sp,
                   insem0, insem1, outsem0, outsem1):
        c = lax.axis_index("c")
        sid = lax.axis_index("s")
        wid = c * NS + sid
        r0 = sid * STRIPE
        insem = (insem0, insem1)
        outsem = (outsem0, outsem1)

        def _zero_row(i, _):
            for j in range(D // 16):
                wbuf[0, i, pl.ds(j * 16, 16)] = jnp.zeros((16,), jnp.float32)
            return 0

        lax.fori_loop(0, CHUNK, _zero_row, 0)
        for g in range(STRIPE // 16):
            denstage[pl.ds(g * 16, 16)] = jnp.zeros((16,), jnp.float32)

        z2d = wbuf.at[0]
        nfull = STRIPE // CHUNK
        for part in range(nfull):
            pltpu.sync_copy(z2d, acc_sp.at[pl.ds(r0 + part * CHUNK, CHUNK)])
        pltpu.sync_copy(z2d.at[pl.ds(0, STRIPE - nfull * CHUNK)],
                        acc_sp.at[pl.ds(r0 + nfull * CHUNK,
                                        STRIPE - nfull * CHUNK)])
        pltpu.sync_copy(denstage, den_sp.at[pl.ds(r0, STRIPE)])

        @pl.when(sid == 0)
        def _zero_tail():
            pltpu.sync_copy(z2d.at[pl.ds(0, TAIL)],
                            acc_sp.at[pl.ds(NS * STRIPE, TAIL)])
            pltpu.sync_copy(denstage.at[pl.ds(0, TAIL)],
                            den_sp.at[pl.ds(NS * STRIPE, TAIL)])

        plsc.subcore_barrier()

        def _issue_in(b, k):
            ci = k * NW + wid
            rl = ci * CHUNK
            pltpu.async_copy(x_hbm.at[pl.ds(base + rl, CHUNK)], xbuf.at[b],
                             insem[b])
            pltpu.async_copy(ex_hbm.at[pl.ds(rl, CHUNK)], exbuf.at[b],
                             insem[b])
            pltpu.async_copy(ids_hbm.at[pl.ds(base + rl, CHUNK)],
                             idbuf.at[b, 0], insem[b])

        def _wait_in(b, k):
            ci = k * NW + wid
            rl = ci * CHUNK
            pltpu.make_async_copy(x_hbm.at[pl.ds(base + rl, CHUNK)],
                                  xbuf.at[b], insem[b]).wait()
            pltpu.make_async_copy(ex_hbm.at[pl.ds(rl, CHUNK)], exbuf.at[b],
                                  insem[b]).wait()
            pltpu.make_async_copy(ids_hbm.at[pl.ds(base + rl, CHUNK)],
                                  idbuf.at[b, 0], insem[b]).wait()

        def _compute(b):
            def _grp(g, _):
                iv = idbuf[b, 0, pl.ds(g * 16, 16)]
                sidbuf[b, 0, pl.ds(g * 16, 16)] = iv
                ev = exbuf[b, pl.ds(g * 16, 16)]
                wexbuf[b, pl.ds(g * 16, 16)] = ev
                for q in range(16):
                    i = g * 16 + q
                    e = ev[q]
                    for j in range(D // 16):
                        wbuf[b, i, pl.ds(j * 16, 16)] = (
                            xbuf[b, i, pl.ds(j * 16, 16)] * e)
                return 0

            lax.fori_loop(0, CHUNK // 16, _grp, 0)

        def _issue_out(b):
            pltpu.async_copy(wbuf.at[b], acc_sp.at[sidbuf.at[b, 0]],
                             outsem[b], add=True)
            pltpu.async_copy(wexbuf.at[b], den_sp.at[sidbuf.at[b, 0]],
                             outsem[b], add=True)

        def _wait_out(b):
            pltpu.make_async_copy(wbuf.at[b], acc_sp.at[sidbuf.at[b, 0]],
                                  outsem[b]).wait()
            pltpu.make_async_copy(wexbuf.at[b], den_sp.at[sidbuf.at[b, 0]],
                                  outsem[b]).wait()

        _issue_in(0, 0)
        _issue_in(1, 1)

        def _pair(p, _):
            k0 = 2 * p
            for b in (0, 1):
                k = k0 + b
                _wait_in(b, k)

                @pl.when(p >= 1)
                def _():
                    _wait_out(b)

                _compute(b)
                _issue_out(b)

                @pl.when((k + 2) * NW + wid < NCH_HALF)
                def _():
                    _issue_in(b, k + 2)
            return 0

        lax.fori_loop(0, NPAIR, _pair, 0)

        @pl.when(62 * NW + wid < NCH_HALF)
        def _epi():
            _wait_in(0, 62)
            _wait_out(0)
            _compute(0)
            _issue_out(0)

        _wait_out(1)
        _wait_out(0)

        plsc.subcore_barrier()
        pltpu.sync_copy(acc_sp.at[pl.ds(r0, STRIPE)],
                        acc_out.at[c, pl.ds(r0, STRIPE)])
        pltpu.sync_copy(den_sp.at[pl.ds(r0, STRIPE)], denstage)
        pltpu.sync_copy(denstage, den_out.at[pl.ds(c * S + r0, STRIPE)])

        @pl.when(sid == 0)
        def _copy_tail():
            pltpu.sync_copy(acc_sp.at[pl.ds(NS * STRIPE, TAIL)],
                            acc_out.at[c, pl.ds(NS * STRIPE, TAIL)])
            pltpu.sync_copy(den_sp.at[pl.ds(NS * STRIPE, TAIL)],
                            denstage.at[pl.ds(0, TAIL)])
            pltpu.sync_copy(denstage.at[pl.ds(0, TAIL)],
                            den_out.at[pl.ds(c * S + NS * STRIPE, TAIL)])

    return _pool_body


@functools.cache
def _pool(h):
    mesh = plsc.VectorSubcoreMesh(
        core_axis_name="c", subcore_axis_name="s",
        num_cores=NC, num_subcores=NS,
    )
    return pl.kernel(
        _make_pool_body(h),
        out_type=[
            jax.ShapeDtypeStruct((NC, S, D), jnp.float32),
            jax.ShapeDtypeStruct((NC * S,), jnp.float32),
        ],
        mesh=mesh,
        scratch_types=[
            pltpu.VMEM((2, CHUNK, D), jnp.float32),
            pltpu.VMEM((2, CHUNK, D), jnp.float32),
            pltpu.VMEM((2, CHUNK), jnp.float32),
            pltpu.VMEM((2, CHUNK), jnp.float32),
            pltpu.VMEM((2, 1, CHUNK), jnp.int32),
            pltpu.VMEM((2, 1, CHUNK), jnp.int32),
            pltpu.VMEM((STRIPE,), jnp.float32),
            pltpu.VMEM_SHARED((S, D), jnp.float32),
            pltpu.VMEM_SHARED((S,), jnp.float32),
            pltpu.SemaphoreType.DMA,
            pltpu.SemaphoreType.DMA,
            pltpu.SemaphoreType.DMA,
            pltpu.SemaphoreType.DMA,
        ],
    )


def _combine_body(acc0_ref, den0_ref, acc1_ref, den1_ref, out_ref):
    d0, d1 = den0_ref[...], den1_ref[...]
    den = (d0[0:S] + d0[S:2 * S] + d1[0:S] + d1[S:2 * S])[:, None]
    a0, a1 = acc0_ref[...], acc1_ref[...]
    w = a0[0] + a0[1] + a1[0] + a1[1]
    safe = jnp.where(den > 0, den, 1.0)
    out_ref[...] = jnp.where(den > 0, w / safe, 0.0)


def _combine(acc0, den0, acc1, den1):
    return pl.pallas_call(
        _combine_body,
        out_shape=jax.ShapeDtypeStruct((S, D), jnp.float32),
    )(acc0, den0, acc1, den1)


def kernel(x, batch, W1, b1, W2, b2):
    ex0 = _scores(x[:N2], W1, b1, W2, b2)
    acc0, den0 = _pool(0)(x, ex0.reshape(N2), batch)
    ex1 = _scores(x[N2:], W1, b1, W2, b2)
    acc1, den1 = _pool(1)(x, ex1.reshape(N2), batch)
    return _combine(acc0, den0, acc1, den1)


# speedup vs baseline: 2.9132x; 1.4374x over previous
import functools

import jax
import jax.numpy as jnp
from jax import lax
from jax.experimental import pallas as pl
from jax.experimental.pallas import tpu as pltpu
from jax.experimental.pallas import tpu_sc as plsc

N = 320000
D = 128
S = 10000
NC = 2
NS = 16
NW = NC * NS
CHUNK = 80
NCH_TOT = N // CHUNK
KPT = NCH_TOT // NW
NPAIR = (KPT - 1) // 2
STRIPE = 624
TAIL = S - NS * STRIPE

RB = 12800


def _scores_body(x_ref, w1_ref, b1_ref, w2_ref, b2_ref, ex_ref):
    h = jnp.tanh(
        jnp.dot(x_ref[...], w1_ref[...], preferred_element_type=jnp.float32)
        + b1_ref[...]
    )
    s = jnp.dot(h, w2_ref[...], preferred_element_type=jnp.float32) + b2_ref[...]
    ex_ref[...] = jnp.exp(s).reshape(1, RB // 128, 128)


def _scores(x, W1, b1, W2, b2):
    return pl.pallas_call(
        _scores_body,
        grid=(N // RB,),
        in_specs=[
            pl.BlockSpec((RB, D), lambda i: (i, 0)),
            pl.BlockSpec((D, D // 2), lambda i: (0, 0)),
            pl.BlockSpec((1, D // 2), lambda i: (0, 0)),
            pl.BlockSpec((D // 2, 1), lambda i: (0, 0)),
            pl.BlockSpec((1, 1), lambda i: (0, 0)),
        ],
        out_specs=pl.BlockSpec((1, RB // 128, 128), lambda i: (i, 0, 0)),
        out_shape=jax.ShapeDtypeStruct((N // RB, RB // 128, 128), jnp.float32),
    )(x, W1, b1.reshape(1, -1), W2, b2.reshape(1, 1))


def _pool_body(x_hbm, ex_hbm, ids_hbm, acc_out, den_out,
               xbuf, wbuf, exbuf, wexbuf, idbuf, sidbuf, denstage,
               acc_sp, den_sp,
               insem0, insem1, outsem0, outsem1):
    c = lax.axis_index("c")
    sid = lax.axis_index("s")
    wid = c * NS + sid
    r0 = sid * STRIPE
    insem = (insem0, insem1)
    outsem = (outsem0, outsem1)

    def _zero_row(i, _):
        for j in range(D // 16):
            wbuf[0, i, pl.ds(j * 16, 16)] = jnp.zeros((16,), jnp.float32)
        return 0

    lax.fori_loop(0, CHUNK, _zero_row, 0)
    for g in range(STRIPE // 16):
        denstage[pl.ds(g * 16, 16)] = jnp.zeros((16,), jnp.float32)

    z2d = wbuf.at[0]
    nfull = STRIPE // CHUNK
    for part in range(nfull):
        pltpu.sync_copy(z2d, acc_sp.at[pl.ds(r0 + part * CHUNK, CHUNK)])
    pltpu.sync_copy(z2d.at[pl.ds(0, STRIPE - nfull * CHUNK)],
                    acc_sp.at[pl.ds(r0 + nfull * CHUNK,
                                    STRIPE - nfull * CHUNK)])
    pltpu.sync_copy(denstage, den_sp.at[pl.ds(r0, STRIPE)])

    @pl.when(sid == 0)
    def _zero_tail():
        pltpu.sync_copy(z2d.at[pl.ds(0, TAIL)],
                        acc_sp.at[pl.ds(NS * STRIPE, TAIL)])
        pltpu.sync_copy(denstage.at[pl.ds(0, TAIL)],
                        den_sp.at[pl.ds(NS * STRIPE, TAIL)])

    plsc.subcore_barrier()

    def _issue_in(b, k):
        ci = k * NW + wid
        r = ci * CHUNK
        pltpu.async_copy(x_hbm.at[pl.ds(r, CHUNK)], xbuf.at[b], insem[b])
        pltpu.async_copy(ex_hbm.at[pl.ds(r, CHUNK)], exbuf.at[b], insem[b])
        pltpu.async_copy(ids_hbm.at[pl.ds(r, CHUNK)], idbuf.at[b, 0],
                         insem[b])

    def _wait_in(b, k):
        ci = k * NW + wid
        r = ci * CHUNK
        pltpu.make_async_copy(x_hbm.at[pl.ds(r, CHUNK)], xbuf.at[b],
                              insem[b]).wait()
        pltpu.make_async_copy(ex_hbm.at[pl.ds(r, CHUNK)], exbuf.at[b],
                              insem[b]).wait()
        pltpu.make_async_copy(ids_hbm.at[pl.ds(r, CHUNK)], idbuf.at[b, 0],
                              insem[b]).wait()

    def _compute(b):
        def _grp(g, _):
            iv = idbuf[b, 0, pl.ds(g * 16, 16)]
            sidbuf[b, 0, pl.ds(g * 16, 16)] = iv
            ev = exbuf[b, pl.ds(g * 16, 16)]
            wexbuf[b, pl.ds(g * 16, 16)] = ev
            for q in range(16):
                i = g * 16 + q
                e = ev[q]
                for j in range(D // 16):
                    wbuf[b, i, pl.ds(j * 16, 16)] = (
                        xbuf[b, i, pl.ds(j * 16, 16)] * e)
            return 0

        lax.fori_loop(0, CHUNK // 16, _grp, 0)

    def _issue_out(b):
        pltpu.async_copy(wbuf.at[b], acc_sp.at[sidbuf.at[b, 0]],
                         outsem[b], add=True)
        pltpu.async_copy(wexbuf.at[b], den_sp.at[sidbuf.at[b, 0]],
                         outsem[b], add=True)

    def _wait_out(b):
        pltpu.make_async_copy(wbuf.at[b], acc_sp.at[sidbuf.at[b, 0]],
                              outsem[b]).wait()
        pltpu.make_async_copy(wexbuf.at[b], den_sp.at[sidbuf.at[b, 0]],
                              outsem[b]).wait()

    _issue_in(0, 0)
    _issue_in(1, 1)

    def _pair(p, _):
        k0 = 2 * p
        for b in (0, 1):
            k = k0 + b
            _wait_in(b, k)

            @pl.when(p >= 1)
            def _():
                _wait_out(b)

            _compute(b)
            _issue_out(b)

            @pl.when(k + 2 <= KPT - 1)
            def _():
                _issue_in(b, k + 2)
        return 0

    lax.fori_loop(0, NPAIR, _pair, 0)

    _wait_in(0, KPT - 1)
    _wait_out(0)
    _compute(0)
    _issue_out(0)
    _wait_out(1)
    _wait_out(0)

    plsc.subcore_barrier()
    pltpu.sync_copy(acc_sp.at[pl.ds(r0, STRIPE)],
                    acc_out.at[c, pl.ds(r0, STRIPE)])
    pltpu.sync_copy(den_sp.at[pl.ds(r0, STRIPE)], denstage)
    pltpu.sync_copy(denstage, den_out.at[pl.ds(c * S + r0, STRIPE)])

    @pl.when(sid == 0)
    def _copy_tail():
        pltpu.sync_copy(acc_sp.at[pl.ds(NS * STRIPE, TAIL)],
                        acc_out.at[c, pl.ds(NS * STRIPE, TAIL)])
        pltpu.sync_copy(den_sp.at[pl.ds(NS * STRIPE, TAIL)],
                        denstage.at[pl.ds(0, TAIL)])
        pltpu.sync_copy(denstage.at[pl.ds(0, TAIL)],
                        den_out.at[pl.ds(c * S + NS * STRIPE, TAIL)])


@functools.cache
def _pool():
    mesh = plsc.VectorSubcoreMesh(
        core_axis_name="c", subcore_axis_name="s",
        num_cores=NC, num_subcores=NS,
    )
    return pl.kernel(
        _pool_body,
        out_type=[
            jax.ShapeDtypeStruct((NC, S, D), jnp.float32),
            jax.ShapeDtypeStruct((NC * S,), jnp.float32),
        ],
        mesh=mesh,
        scratch_types=[
            pltpu.VMEM((2, CHUNK, D), jnp.float32),
            pltpu.VMEM((2, CHUNK, D), jnp.float32),
            pltpu.VMEM((2, CHUNK), jnp.float32),
            pltpu.VMEM((2, CHUNK), jnp.float32),
            pltpu.VMEM((2, 1, CHUNK), jnp.int32),
            pltpu.VMEM((2, 1, CHUNK), jnp.int32),
            pltpu.VMEM((STRIPE,), jnp.float32),
            pltpu.VMEM_SHARED((S, D), jnp.float32),
            pltpu.VMEM_SHARED((S,), jnp.float32),
            pltpu.SemaphoreType.DMA,
            pltpu.SemaphoreType.DMA,
            pltpu.SemaphoreType.DMA,
            pltpu.SemaphoreType.DMA,
        ],
    )


def _combine_body(acc_ref, den_ref, out_ref):
    den_all = den_ref[...]
    den = (den_all[0:S] + den_all[S:2 * S])[:, None]
    w = acc_ref[...][0] + acc_ref[...][1]
    safe = jnp.where(den > 0, den, 1.0)
    out_ref[...] = jnp.where(den > 0, w / safe, 0.0)


def _combine(acc, den):
    return pl.pallas_call(
        _combine_body,
        out_shape=jax.ShapeDtypeStruct((S, D), jnp.float32),
    )(acc, den)


def kernel(x, batch, W1, b1, W2, b2):
    ex = _scores(x, W1, b1, W2, b2)
    acc, den = _pool()(x, ex.reshape(N), batch)
    return _combine(acc, den)


# speedup vs baseline: 3.0002x; 1.0299x over previous
import functools

import jax
import jax.numpy as jnp
from jax import lax
from jax.experimental import pallas as pl
from jax.experimental.pallas import tpu as pltpu
from jax.experimental.pallas import tpu_sc as plsc

N = 320000
D = 128
S = 10000
NC = 2
NS = 16
NW = NC * NS
CHUNK = 80
NCH_TOT = N // CHUNK
KPT = NCH_TOT // NW
NPAIR = (KPT - 1) // 2
STRIPE = 624
TAIL = S - NS * STRIPE

RB = 32000


def _scores_body(x_ref, w1_ref, b1_ref, w2_ref, b2_ref, ex_ref):
    h = jnp.tanh(
        jnp.dot(x_ref[...], w1_ref[...], preferred_element_type=jnp.float32)
        + b1_ref[...]
    )
    s = jnp.dot(h, w2_ref[...], preferred_element_type=jnp.float32) + b2_ref[...]
    ex_ref[...] = jnp.exp(s).reshape(1, RB // 128, 128)


def _scores(x, W1, b1, W2, b2):
    return pl.pallas_call(
        _scores_body,
        grid=(N // RB,),
        in_specs=[
            pl.BlockSpec((RB, D), lambda i: (i, 0)),
            pl.BlockSpec((D, D // 2), lambda i: (0, 0)),
            pl.BlockSpec((1, D // 2), lambda i: (0, 0)),
            pl.BlockSpec((D // 2, 1), lambda i: (0, 0)),
            pl.BlockSpec((1, 1), lambda i: (0, 0)),
        ],
        out_specs=pl.BlockSpec((1, RB // 128, 128), lambda i: (i, 0, 0)),
        out_shape=jax.ShapeDtypeStruct((N // RB, RB // 128, 128), jnp.float32),
    )(x, W1, b1.reshape(1, -1), W2, b2.reshape(1, 1))


def _pool_body(x_hbm, ex_hbm, ids_hbm, acc_out, den_out,
               xbuf, wbuf, exbuf, wexbuf, idbuf, sidbuf, denstage,
               acc_sp, den_sp,
               insem0, insem1, outsem0, outsem1):
    c = lax.axis_index("c")
    sid = lax.axis_index("s")
    wid = c * NS + sid
    r0 = sid * STRIPE
    insem = (insem0, insem1)
    outsem = (outsem0, outsem1)

    def _zero_row(i, _):
        for j in range(D // 16):
            wbuf[0, i, pl.ds(j * 16, 16)] = jnp.zeros((16,), jnp.float32)
        return 0

    lax.fori_loop(0, CHUNK, _zero_row, 0)
    for g in range(STRIPE // 16):
        denstage[pl.ds(g * 16, 16)] = jnp.zeros((16,), jnp.float32)

    z2d = wbuf.at[0]
    nfull = STRIPE // CHUNK
    for part in range(nfull):
        pltpu.sync_copy(z2d, acc_sp.at[pl.ds(r0 + part * CHUNK, CHUNK)])
    pltpu.sync_copy(z2d.at[pl.ds(0, STRIPE - nfull * CHUNK)],
                    acc_sp.at[pl.ds(r0 + nfull * CHUNK,
                                    STRIPE - nfull * CHUNK)])
    pltpu.sync_copy(denstage, den_sp.at[pl.ds(r0, STRIPE)])

    @pl.when(sid == 0)
    def _zero_tail():
        pltpu.sync_copy(z2d.at[pl.ds(0, TAIL)],
                        acc_sp.at[pl.ds(NS * STRIPE, TAIL)])
        pltpu.sync_copy(denstage.at[pl.ds(0, TAIL)],
                        den_sp.at[pl.ds(NS * STRIPE, TAIL)])

    plsc.subcore_barrier()

    def _issue_in(b, k):
        ci = k * NW + wid
        r = ci * CHUNK
        pltpu.async_copy(x_hbm.at[pl.ds(r, CHUNK)], xbuf.at[b], insem[b])
        pltpu.async_copy(ex_hbm.at[pl.ds(r, CHUNK)], exbuf.at[b], insem[b])
        pltpu.async_copy(ids_hbm.at[pl.ds(r, CHUNK)], idbuf.at[b, 0],
                         insem[b])

    def _wait_in(b, k):
        ci = k * NW + wid
        r = ci * CHUNK
        pltpu.make_async_copy(x_hbm.at[pl.ds(r, CHUNK)], xbuf.at[b],
                              insem[b]).wait()
        pltpu.make_async_copy(ex_hbm.at[pl.ds(r, CHUNK)], exbuf.at[b],
                              insem[b]).wait()
        pltpu.make_async_copy(ids_hbm.at[pl.ds(r, CHUNK)], idbuf.at[b, 0],
                              insem[b]).wait()

    def _compute(b):
        def _grp(g, _):
            iv = idbuf[b, 0, pl.ds(g * 16, 16)]
            sidbuf[b, 0, pl.ds(g * 16, 16)] = iv
            ev = exbuf[b, pl.ds(g * 16, 16)]
            wexbuf[b, pl.ds(g * 16, 16)] = ev
            for q in range(16):
                i = g * 16 + q
                e = ev[q]
                for j in range(D // 16):
                    wbuf[b, i, pl.ds(j * 16, 16)] = (
                        xbuf[b, i, pl.ds(j * 16, 16)] * e)
            return 0

        lax.fori_loop(0, CHUNK // 16, _grp, 0)

    def _issue_out(b):
        pltpu.async_copy(wbuf.at[b], acc_sp.at[sidbuf.at[b, 0]],
                         outsem[b], add=True)
        pltpu.async_copy(wexbuf.at[b], den_sp.at[sidbuf.at[b, 0]],
                         outsem[b], add=True)

    def _wait_out(b):
        pltpu.make_async_copy(wbuf.at[b], acc_sp.at[sidbuf.at[b, 0]],
                              outsem[b]).wait()
        pltpu.make_async_copy(wexbuf.at[b], den_sp.at[sidbuf.at[b, 0]],
                              outsem[b]).wait()

    _issue_in(0, 0)
    _issue_in(1, 1)

    def _pair(p, _):
        k0 = 2 * p
        for b in (0, 1):
            k = k0 + b
            _wait_in(b, k)

            @pl.when(p >= 1)
            def _():
                _wait_out(b)

            _compute(b)
            _issue_out(b)

            @pl.when(k + 2 <= KPT - 1)
            def _():
                _issue_in(b, k + 2)
        return 0

    lax.fori_loop(0, NPAIR, _pair, 0)

    _wait_in(0, KPT - 1)
    _wait_out(0)
    _compute(0)
    _issue_out(0)
    _wait_out(1)
    _wait_out(0)

    plsc.subcore_barrier()
    pltpu.sync_copy(acc_sp.at[pl.ds(r0, STRIPE)],
                    acc_out.at[c, pl.ds(r0, STRIPE)])
    pltpu.sync_copy(den_sp.at[pl.ds(r0, STRIPE)], denstage)
    pltpu.sync_copy(denstage, den_out.at[pl.ds(c * S + r0, STRIPE)])

    @pl.when(sid == 0)
    def _copy_tail():
        pltpu.sync_copy(acc_sp.at[pl.ds(NS * STRIPE, TAIL)],
                        acc_out.at[c, pl.ds(NS * STRIPE, TAIL)])
        pltpu.sync_copy(den_sp.at[pl.ds(NS * STRIPE, TAIL)],
                        denstage.at[pl.ds(0, TAIL)])
        pltpu.sync_copy(denstage.at[pl.ds(0, TAIL)],
                        den_out.at[pl.ds(c * S + NS * STRIPE, TAIL)])


@functools.cache
def _pool():
    mesh = plsc.VectorSubcoreMesh(
        core_axis_name="c", subcore_axis_name="s",
        num_cores=NC, num_subcores=NS,
    )
    return pl.kernel(
        _pool_body,
        out_type=[
            jax.ShapeDtypeStruct((NC, S, D), jnp.float32),
            jax.ShapeDtypeStruct((NC * S,), jnp.float32),
        ],
        mesh=mesh,
        scratch_types=[
            pltpu.VMEM((2, CHUNK, D), jnp.float32),
            pltpu.VMEM((2, CHUNK, D), jnp.float32),
            pltpu.VMEM((2, CHUNK), jnp.float32),
            pltpu.VMEM((2, CHUNK), jnp.float32),
            pltpu.VMEM((2, 1, CHUNK), jnp.int32),
            pltpu.VMEM((2, 1, CHUNK), jnp.int32),
            pltpu.VMEM((STRIPE,), jnp.float32),
            pltpu.VMEM_SHARED((S, D), jnp.float32),
            pltpu.VMEM_SHARED((S,), jnp.float32),
            pltpu.SemaphoreType.DMA,
            pltpu.SemaphoreType.DMA,
            pltpu.SemaphoreType.DMA,
            pltpu.SemaphoreType.DMA,
        ],
    )


def _combine_body(acc_ref, den_ref, out_ref):
    den_all = den_ref[...]
    den = (den_all[0:S] + den_all[S:2 * S])[:, None]
    w = acc_ref[...][0] + acc_ref[...][1]
    safe = jnp.where(den > 0, den, 1.0)
    out_ref[...] = jnp.where(den > 0, w / safe, 0.0)


def _combine(acc, den):
    return pl.pallas_call(
        _combine_body,
        out_shape=jax.ShapeDtypeStruct((S, D), jnp.float32),
    )(acc, den)


def kernel(x, batch, W1, b1, W2, b2):
    ex = _scores(x, W1, b1, W2, b2)
    acc, den = _pool()(x, ex.reshape(N), batch)
    return _combine(acc, den)
